# baseline (speedup 1.0000x reference)
import functools

import jax
import jax.numpy as jnp
from jax import lax
from jax.experimental import pallas as pl
from jax.experimental.pallas import tpu as pltpu
from jax.experimental.pallas import tpu_sc as plsc

_N = 10000
_E = 320000
_D = 128
_NC = 2
_NS = 16
_NW = _NC * _NS
_EPW = _E // _NW
_C = 80
_NCHUNK = _EPW // _C
_NB = 3
_RPT = _N // _NS
_WBR = 624
_TC_BR = 1000


def _tanh_body(x_ref, o_ref):
    o_ref[...] = jnp.tanh(x_ref[...])


def _add_body(a_ref, b_ref, o_ref):
    o_ref[...] = a_ref[0] + b_ref[0]


_sc_mesh = plsc.VectorSubcoreMesh(core_axis_name="c", subcore_axis_name="s")


@functools.partial(
    pl.kernel,
    out_type=jax.ShapeDtypeStruct((_NC, _N, _D), jnp.float32),
    mesh=_sc_mesh,
    compiler_params=pltpu.CompilerParams(needs_layout_passes=False),
    scratch_types=[
        pltpu.VMEM((_EPW,), jnp.int32),
        [pltpu.VMEM((_C,), jnp.int32)] * _NB,
        [pltpu.VMEM((_C,), jnp.int32)] * _NB,
        [pltpu.VMEM((_C,), jnp.int32)] * _NB,
        [pltpu.VMEM((_C,), jnp.int32)] * _NB,
        [pltpu.VMEM((_C,), jnp.int32)] * _NB,
        [pltpu.VMEM((_C, _D), jnp.float32)] * _NB,
        pltpu.VMEM_SHARED((_N, _D), jnp.float32),
        [pltpu.SemaphoreType.DMA] * _NB,
        [pltpu.SemaphoreType.DMA] * _NB,
        [pltpu.SemaphoreType.DMA] * _NB,
    ],
)
def _sc_scatter(t_hbm, widx_hbm, wflat_hbm, out_hbm,
                widx_v, srcs, dsts, wps, shs, wvs, rowss, acc_sh, gw, gt, sc):
    c = lax.axis_index("c")
    s = lax.axis_index("s")
    wid = c * _NS + s

    pltpu.sync_copy(widx_hbm.at[pl.ds(wid * _EPW, _EPW)], widx_v)

    def _zrow(e, carry):
        z = jnp.zeros((16,), jnp.float32)
        for j in range(_D // 16):
            rowss[0][e, pl.ds(j * 16, 16)] = z
        return carry

    lax.fori_loop(0, _C, _zrow, 0)
    zbase = s * _RPT
    for r in range(_RPT // _C):
        pltpu.sync_copy(rowss[0], acc_sh.at[pl.ds(zbase + r * _C, _C)])
    _rem = _RPT - (_RPT // _C) * _C
    pltpu.sync_copy(rowss[0].at[pl.ds(0, _rem)],
                    acc_sh.at[pl.ds(zbase + (_RPT // _C) * _C, _rem)])
    plsc.subcore_barrier()

    def _issue(k, b):
        for i in range(_C // 16):
            wv16 = widx_v[pl.ds(k * _C + i * 16, 16)]
            s16 = wv16 // _N
            sl = pl.ds(i * 16, 16)
            srcs[b][sl] = s16
            dsts[b][sl] = wv16 - s16 * _N
            wps[b][sl] = wv16 >> 1
            shs[b][sl] = 16 - ((wv16 & 1) << 4)
        pltpu.async_copy(wflat_hbm.at[wps[b]], wvs[b], gw[b])
        pltpu.async_copy(t_hbm.at[srcs[b]], rowss[b], gt[b])

    def _wait_gathers(k, b):
        pltpu.make_async_copy(wflat_hbm.at[wps[b]], wvs[b], gw[b]).wait()
        pltpu.make_async_copy(t_hbm.at[srcs[b]], rowss[b], gt[b]).wait()

    def _scale(b):
        def _srow(e4, cc):
            for v in range(4):
                e = e4 * 4 + v
                eidx = jnp.full((16,), e, jnp.int32)
                wu = plsc.load_gather(wvs[b], [eidx])
                sh = plsc.load_gather(shs[b], [eidx])
                wbits = (wu << sh) & jnp.int32(-65536)
                wsc = plsc.bitcast(wbits, jnp.float32)
                for j in range(_D // 16):
                    sl = pl.ds(j * 16, 16)
                    rowss[b][e, sl] = rowss[b][e, sl] * wsc
            return cc

        lax.fori_loop(0, _C // 4, _srow, 0)

    def _scatter(k, b):
        pltpu.async_copy(rowss[b], acc_sh.at[dsts[b]], sc[b], add=True)

    def _wait_scatter(k, b):
        pltpu.make_async_copy(rowss[b], acc_sh.at[dsts[b]], sc[b]).wait()

    _issue(0, 0)

    def _group(i, carry):
        for u in range(_NB):
            k = i * _NB + u
            b = u
            b1 = (u + 1) % _NB

            @pl.when(k <= _NCHUNK - 1)
            def _body():
                @pl.when(jnp.logical_and(k >= 2, k <= _NCHUNK - 2))
                def _free():
                    _wait_scatter(k - 2, b1)

                @pl.when(k <= _NCHUNK - 2)
                def _prefetch():
                    _issue(k + 1, b1)

                _wait_gathers(k, b)
                _scale(b)
                _scatter(k, b)

        return carry

    lax.fori_loop(0, (_NCHUNK + _NB - 1) // _NB, _group, 0)
    for kk in range(_NCHUNK - 3, _NCHUNK):
        _wait_scatter(kk, kk % _NB)

    plsc.subcore_barrier()

    wb = s * _WBR
    pltpu.sync_copy(acc_sh.at[pl.ds(wb, _WBR)],
                    out_hbm.at[c, pl.ds(wb, _WBR)])

    @pl.when(s == _NS - 1)
    def _tail():
        pltpu.sync_copy(acc_sh.at[pl.ds(_NS * _WBR, _N - _NS * _WBR)],
                        out_hbm.at[c, pl.ds(_NS * _WBR, _N - _NS * _WBR)])


def kernel(values, edge_index, w):
    widx = edge_index[0] * _N + edge_index[1]
    wflat = jax.lax.bitcast_convert_type(
        w.astype(jnp.bfloat16).reshape(_N * _N // 2, 2), jnp.int32)

    t = pl.pallas_call(
        _tanh_body,
        grid=(_N // _TC_BR,),
        in_specs=[pl.BlockSpec((_TC_BR, _D), lambda i: (i, 0))],
        out_specs=pl.BlockSpec((_TC_BR, _D), lambda i: (i, 0)),
        out_shape=jax.ShapeDtypeStruct((_N, _D), jnp.float32),
    )(values)

    partials = _sc_scatter(t, widx, wflat)

    out = pl.pallas_call(
        _add_body,
        grid=(_N // _TC_BR,),
        in_specs=[
            pl.BlockSpec((1, _TC_BR, _D), lambda i: (0, i, 0)),
            pl.BlockSpec((1, _TC_BR, _D), lambda i: (1, i, 0)),
        ],
        out_specs=pl.BlockSpec((_TC_BR, _D), lambda i: (i, 0)),
        out_shape=jax.ShapeDtypeStruct((_N, _D), jnp.float32),
    )(partials, partials)
    return out


# speedup vs baseline: 34.6021x; 34.6021x over previous
import functools

import jax
import jax.numpy as jnp
from jax import lax
from jax.experimental import pallas as pl
from jax.experimental.pallas import tpu as pltpu
from jax.experimental.pallas import tpu_sc as plsc

_N = 10000
_E = 320000
_D = 128
_NC = 2
_NS = 16
_NW = _NC * _NS
_EPW = _E // _NW
_C = 80
_NCHUNK = _EPW // _C
_NB = 3
_RPT = _N // _NS
_WBR = 624
_TC_BR = 1000


def _tanh_body(x_ref, o_ref):
    o_ref[...] = jnp.tanh(x_ref[...])


def _add_body(a_ref, b_ref, o_ref):
    o_ref[...] = a_ref[0] + b_ref[0]


_sc_mesh = plsc.VectorSubcoreMesh(core_axis_name="c", subcore_axis_name="s")


@functools.partial(
    pl.kernel,
    out_type=jax.ShapeDtypeStruct((_NC, _N, _D), jnp.float32),
    mesh=_sc_mesh,
    compiler_params=pltpu.CompilerParams(needs_layout_passes=False),
    scratch_types=[
        pltpu.VMEM((_EPW,), jnp.int32),
        [pltpu.VMEM((_C,), jnp.int32)] * _NB,
        [pltpu.VMEM((_C,), jnp.int32)] * _NB,
        [pltpu.VMEM((_C,), jnp.int32)] * _NB,
        [pltpu.VMEM((_C,), jnp.int32)] * _NB,
        [pltpu.VMEM((_C,), jnp.int32)] * _NB,
        [pltpu.VMEM((_C, _D), jnp.float32)] * _NB,
        pltpu.VMEM_SHARED((_N, _D), jnp.float32),
        [pltpu.SemaphoreType.DMA] * _NB,
        [pltpu.SemaphoreType.DMA] * _NB,
        [pltpu.SemaphoreType.DMA] * _NB,
    ],
)
def _sc_scatter(t_hbm, widx_hbm, wflat_hbm, out_hbm,
                widx_v, srcs, dsts, wps, shs, wvs, rowss, acc_sh, gw, gt, sc):
    c = lax.axis_index("c")
    s = lax.axis_index("s")
    wid = c * _NS + s

    pltpu.sync_copy(widx_hbm.at[pl.ds(wid * _EPW, _EPW)], widx_v)

    def _zrow(e, carry):
        z = jnp.zeros((16,), jnp.float32)
        for j in range(_D // 16):
            rowss[0][e, pl.ds(j * 16, 16)] = z
        return carry

    lax.fori_loop(0, _C, _zrow, 0)
    zbase = s * _RPT
    for r in range(_RPT // _C):
        pltpu.sync_copy(rowss[0], acc_sh.at[pl.ds(zbase + r * _C, _C)])
    _rem = _RPT - (_RPT // _C) * _C
    pltpu.sync_copy(rowss[0].at[pl.ds(0, _rem)],
                    acc_sh.at[pl.ds(zbase + (_RPT // _C) * _C, _rem)])
    plsc.subcore_barrier()

    def _issue(k, b):
        for i in range(_C // 16):
            wv16 = widx_v[pl.ds(k * _C + i * 16, 16)]
            s16 = wv16 // _N
            sl = pl.ds(i * 16, 16)
            srcs[b][sl] = s16
            dsts[b][sl] = wv16 - s16 * _N
            h16 = (wv16 >= _N * _N // 2).astype(jnp.int32)
            wps[b][sl] = wv16 - h16 * (_N * _N // 2)
            shs[b][sl] = 16 - (h16 << 4)
        pltpu.async_copy(wflat_hbm.at[wps[b]], wvs[b], gw[b])
        pltpu.async_copy(t_hbm.at[srcs[b]], rowss[b], gt[b])

    def _wait_gathers(k, b):
        pltpu.make_async_copy(wflat_hbm.at[wps[b]], wvs[b], gw[b]).wait()
        pltpu.make_async_copy(t_hbm.at[srcs[b]], rowss[b], gt[b]).wait()

    def _scale(b):
        def _srow(e4, cc):
            for v in range(4):
                e = e4 * 4 + v
                eidx = jnp.full((16,), e, jnp.int32)
                wu = plsc.load_gather(wvs[b], [eidx])
                sh = plsc.load_gather(shs[b], [eidx])
                wbits = (wu << sh) & jnp.int32(-65536)
                wsc = plsc.bitcast(wbits, jnp.float32)
                for j in range(_D // 16):
                    sl = pl.ds(j * 16, 16)
                    rowss[b][e, sl] = rowss[b][e, sl] * wsc
            return cc

        lax.fori_loop(0, _C // 4, _srow, 0)

    def _scatter(k, b):
        pltpu.async_copy(rowss[b], acc_sh.at[dsts[b]], sc[b], add=True)

    def _wait_scatter(k, b):
        pltpu.make_async_copy(rowss[b], acc_sh.at[dsts[b]], sc[b]).wait()

    _issue(0, 0)

    def _group(i, carry):
        for u in range(_NB):
            k = i * _NB + u
            b = u
            b1 = (u + 1) % _NB

            @pl.when(k <= _NCHUNK - 1)
            def _body():
                @pl.when(jnp.logical_and(k >= 2, k <= _NCHUNK - 2))
                def _free():
                    _wait_scatter(k - 2, b1)

                @pl.when(k <= _NCHUNK - 2)
                def _prefetch():
                    _issue(k + 1, b1)

                _wait_gathers(k, b)
                _scale(b)
                _scatter(k, b)

        return carry

    lax.fori_loop(0, (_NCHUNK + _NB - 1) // _NB, _group, 0)
    for kk in range(_NCHUNK - 3, _NCHUNK):
        _wait_scatter(kk, kk % _NB)

    plsc.subcore_barrier()

    wb = s * _WBR
    pltpu.sync_copy(acc_sh.at[pl.ds(wb, _WBR)],
                    out_hbm.at[c, pl.ds(wb, _WBR)])

    @pl.when(s == _NS - 1)
    def _tail():
        pltpu.sync_copy(acc_sh.at[pl.ds(_NS * _WBR, _N - _NS * _WBR)],
                        out_hbm.at[c, pl.ds(_NS * _WBR, _N - _NS * _WBR)])


def kernel(values, edge_index, w):
    widx = edge_index[0] * _N + edge_index[1]
    lo = jax.lax.bitcast_convert_type(
        w[:_N // 2].astype(jnp.bfloat16), jnp.uint16).astype(jnp.uint32)
    hi = jax.lax.bitcast_convert_type(
        w[_N // 2:].astype(jnp.bfloat16), jnp.uint16).astype(jnp.uint32)
    wflat = jax.lax.bitcast_convert_type(
        lo | (hi << 16), jnp.int32).reshape(_N * _N // 2)

    t = pl.pallas_call(
        _tanh_body,
        grid=(_N // _TC_BR,),
        in_specs=[pl.BlockSpec((_TC_BR, _D), lambda i: (i, 0))],
        out_specs=pl.BlockSpec((_TC_BR, _D), lambda i: (i, 0)),
        out_shape=jax.ShapeDtypeStruct((_N, _D), jnp.float32),
    )(values)

    partials = _sc_scatter(t, widx, wflat)

    out = pl.pallas_call(
        _add_body,
        grid=(_N // _TC_BR,),
        in_specs=[
            pl.BlockSpec((1, _TC_BR, _D), lambda i: (0, i, 0)),
            pl.BlockSpec((1, _TC_BR, _D), lambda i: (1, i, 0)),
        ],
        out_specs=pl.BlockSpec((_TC_BR, _D), lambda i: (i, 0)),
        out_shape=jax.ShapeDtypeStruct((_N, _D), jnp.float32),
    )(partials, partials)
    return out


# speedup vs baseline: 57.0288x; 1.6481x over previous
import functools

import jax
import jax.numpy as jnp
from jax import lax
from jax.experimental import pallas as pl
from jax.experimental.pallas import tpu as pltpu
from jax.experimental.pallas import tpu_sc as plsc

_N = 10000
_E = 320000
_D = 128
_NC = 2
_NS = 16
_NW = _NC * _NS
_EPW = _E // _NW
_C = 80
_NCHUNK = _EPW // _C
_NB = 3
_RPT = _N // _NS
_WBR = 624
_TC_BR = 1000


def _tanh_body(x_ref, o_ref):
    o_ref[...] = jnp.tanh(x_ref[...])


def _add_body(a_ref, b_ref, o_ref):
    o_ref[...] = a_ref[0] + b_ref[0]


_sc_mesh = plsc.VectorSubcoreMesh(core_axis_name="c", subcore_axis_name="s")


@functools.partial(
    pl.kernel,
    out_type=jax.ShapeDtypeStruct((_NC, _N, _D), jnp.float32),
    mesh=_sc_mesh,
    compiler_params=pltpu.CompilerParams(needs_layout_passes=False),
    scratch_types=[
        pltpu.VMEM((_EPW,), jnp.int32),
        [pltpu.VMEM((_C,), jnp.int32)] * _NB,
        [pltpu.VMEM((_C,), jnp.int32)] * _NB,
        [pltpu.VMEM((_C,), jnp.int32)] * _NB,
        [pltpu.VMEM((_C,), jnp.float32)] * _NB,
        [pltpu.VMEM((_C, _D), jnp.float32)] * _NB,
        pltpu.VMEM_SHARED((_N, _D), jnp.float32),
        [pltpu.SemaphoreType.DMA] * _NB,
        [pltpu.SemaphoreType.DMA] * _NB,
        [pltpu.SemaphoreType.DMA] * _NB,
    ],
)
def _sc_scatter(t_hbm, widx_hbm, wflat_hbm, out_hbm,
                widx_v, srcs, dsts, wps, wvs, rowss, acc_sh, gw, gt, sc):
    c = lax.axis_index("c")
    s = lax.axis_index("s")
    wid = c * _NS + s

    pltpu.sync_copy(widx_hbm.at[pl.ds(wid * _EPW, _EPW)], widx_v)

    def _zrow(e, carry):
        z = jnp.zeros((16,), jnp.float32)
        for j in range(_D // 16):
            rowss[0][e, pl.ds(j * 16, 16)] = z
        return carry

    lax.fori_loop(0, _C, _zrow, 0)
    zbase = s * _RPT
    for r in range(_RPT // _C):
        pltpu.sync_copy(rowss[0], acc_sh.at[pl.ds(zbase + r * _C, _C)])
    _rem = _RPT - (_RPT // _C) * _C
    pltpu.sync_copy(rowss[0].at[pl.ds(0, _rem)],
                    acc_sh.at[pl.ds(zbase + (_RPT // _C) * _C, _rem)])
    plsc.subcore_barrier()

    def _issue(k, b):
        for i in range(_C // 16):
            wv16 = widx_v[pl.ds(k * _C + i * 16, 16)]
            s16 = wv16 // _N
            d16 = wv16 - s16 * _N
            sl = pl.ds(i * 16, 16)
            srcs[b][sl] = s16
            dsts[b][sl] = d16
            wps[b][sl] = ((s16 >> 3) * (79 * 1024) + ((d16 >> 7) << 10)
                          + ((s16 & 7) << 7) + (d16 & 127))
        pltpu.async_copy(wflat_hbm.at[wps[b]], wvs[b], gw[b])
        pltpu.async_copy(t_hbm.at[srcs[b]], rowss[b], gt[b])

    def _wait_gathers(k, b):
        pltpu.make_async_copy(wflat_hbm.at[wps[b]], wvs[b], gw[b]).wait()
        pltpu.make_async_copy(t_hbm.at[srcs[b]], rowss[b], gt[b]).wait()

    def _scale(b):
        def _srow(e4, cc):
            for v in range(4):
                e = e4 * 4 + v
                eidx = jnp.full((16,), e, jnp.int32)
                wsc = plsc.load_gather(wvs[b], [eidx])
                for j in range(_D // 16):
                    sl = pl.ds(j * 16, 16)
                    rowss[b][e, sl] = rowss[b][e, sl] * wsc
            return cc

        lax.fori_loop(0, _C // 4, _srow, 0)

    def _scatter(k, b):
        pltpu.async_copy(rowss[b], acc_sh.at[dsts[b]], sc[b], add=True)

    def _wait_scatter(k, b):
        pltpu.make_async_copy(rowss[b], acc_sh.at[dsts[b]], sc[b]).wait()

    _issue(0, 0)

    def _group(i, carry):
        for u in range(_NB):
            k = i * _NB + u
            b = u
            b1 = (u + 1) % _NB

            @pl.when(k <= _NCHUNK - 1)
            def _body():
                @pl.when(jnp.logical_and(k >= 2, k <= _NCHUNK - 2))
                def _free():
                    _wait_scatter(k - 2, b1)

                @pl.when(k <= _NCHUNK - 2)
                def _prefetch():
                    _issue(k + 1, b1)

                _wait_gathers(k, b)
                _scale(b)
                _scatter(k, b)

        return carry

    lax.fori_loop(0, (_NCHUNK + _NB - 1) // _NB, _group, 0)
    for kk in range(_NCHUNK - 3, _NCHUNK):
        _wait_scatter(kk, kk % _NB)

    plsc.subcore_barrier()

    wb = s * _WBR
    pltpu.sync_copy(acc_sh.at[pl.ds(wb, _WBR)],
                    out_hbm.at[c, pl.ds(wb, _WBR)])

    @pl.when(s == _NS - 1)
    def _tail():
        pltpu.sync_copy(acc_sh.at[pl.ds(_NS * _WBR, _N - _NS * _WBR)],
                        out_hbm.at[c, pl.ds(_NS * _WBR, _N - _NS * _WBR)])


def kernel(values, edge_index, w):
    widx = edge_index[0] * _N + edge_index[1]
    wpad = jnp.pad(w, ((0, 0), (0, 112)))
    wflat = (wpad.reshape(_N // 8, 8, 79, 128)
             .transpose(0, 2, 1, 3)
             .reshape(_N // 8 * 79 * 8 * 128))

    t = pl.pallas_call(
        _tanh_body,
        grid=(_N // _TC_BR,),
        in_specs=[pl.BlockSpec((_TC_BR, _D), lambda i: (i, 0))],
        out_specs=pl.BlockSpec((_TC_BR, _D), lambda i: (i, 0)),
        out_shape=jax.ShapeDtypeStruct((_N, _D), jnp.float32),
    )(values)

    partials = _sc_scatter(t, widx, wflat)

    out = pl.pallas_call(
        _add_body,
        grid=(_N // _TC_BR,),
        in_specs=[
            pl.BlockSpec((1, _TC_BR, _D), lambda i: (0, i, 0)),
            pl.BlockSpec((1, _TC_BR, _D), lambda i: (1, i, 0)),
        ],
        out_specs=pl.BlockSpec((_TC_BR, _D), lambda i: (i, 0)),
        out_shape=jax.ShapeDtypeStruct((_N, _D), jnp.float32),
    )(partials, partials)
    return out
